# baseline (speedup 1.0000x reference)
import functools

import jax
import jax.numpy as jnp
from jax import lax
from jax.experimental import pallas as pl
from jax.experimental.pallas import tpu as pltpu
from jax.experimental.pallas import tpu_sc as plsc

_E = 8
_K = 2
_D = 1024
_DFF = 2048
_T = 2048
_A = _T * _K
_M = 128
_NT = _A // _M + _E
_P = _NT * _M
_MF = float(_M)


def _router_body(hs_ref, gw_ref, logits_ref, p0_ref, p1_ref, wts_ref,
                 te_ref, act_ref, xi_ref):
    hs = hs_ref[...]
    gw = gw_ref[...]
    logits = lax.dot_general(hs, gw, (((1,), (1,)), ((), ())),
                             preferred_element_type=jnp.float32)
    logits_ref[...] = logits
    p = jax.nn.softmax(logits, axis=-1)
    iota = lax.broadcasted_iota(jnp.int32, p.shape, 1)
    m1 = jnp.max(p, axis=1, keepdims=True)
    i1 = jnp.min(jnp.where(p == m1, iota, _E), axis=1, keepdims=True)
    p2 = jnp.where(iota == i1, -1.0, p)
    m2 = jnp.max(p2, axis=1, keepdims=True)
    i2 = jnp.min(jnp.where(p2 == m2, iota, _E), axis=1, keepdims=True)
    den = m1 + m2
    wts_ref[...] = jnp.concatenate([m1 / den, m2 / den], axis=1)

    onehot1 = (iota == i1)
    onehot2 = (iota == i2)
    mask = jnp.where(onehot1 | onehot2, 1.0, 0.0)
    csum = mask
    s = 1
    while s < _T:
        shifted = jnp.concatenate(
            [jnp.zeros((s, _E), jnp.float32), csum[:_T - s, :]], axis=0)
        csum = csum + shifted
        s *= 2
    rank = csum - mask
    counts = csum[_T - 1:_T, :]

    tiles = jnp.floor((counts + (_MF - 1.0)) * (1.0 / _MF))
    tend = tiles
    s = 1
    while s < _E:
        shifted = jnp.concatenate(
            [jnp.zeros((1, s), jnp.float32), tend[:, :_E - s]], axis=1)
        tend = tend + shifted
        s *= 2
    tstart = tend - tiles
    padded_off = tstart * _MF

    slot = padded_off + rank
    p0_ref[...] = jnp.sum(jnp.where(onehot1, slot, 0.0), axis=1,
                          keepdims=True).astype(jnp.int32)
    p1_ref[...] = jnp.sum(jnp.where(onehot2, slot, 0.0), axis=1,
                          keepdims=True).astype(jnp.int32)

    total = jnp.sum(jnp.where(iota[0:1, :] == _E - 1, tend, 0.0),
                    axis=1, keepdims=True)
    j_iota = lax.broadcasted_iota(jnp.int32, (_NT, 1), 0).astype(jnp.float32)
    te_raw = jnp.sum(
        jnp.where(tend <= j_iota, 1.0, 0.0), axis=1, keepdims=True)
    act = jnp.where(j_iota < total, 1, 0).astype(jnp.int32)
    last = jnp.maximum(total - 1.0, 0.0)
    te_last = jnp.sum(jnp.where(tend <= last, 1.0, 0.0), axis=1,
                      keepdims=True)
    te = jnp.where(act != 0, jnp.minimum(te_raw, float(_E - 1)), te_last)
    te_ref[...] = te.astype(jnp.int32)
    act_ref[...] = act
    xi_ref[...] = jnp.where(act != 0, j_iota, last).astype(jnp.int32)


def _router(hs2d, gate_w):
    return pl.pallas_call(
        _router_body,
        out_shape=(
            jax.ShapeDtypeStruct((_T, _E), jnp.float32),
            jax.ShapeDtypeStruct((_T, 1), jnp.int32),
            jax.ShapeDtypeStruct((_T, 1), jnp.int32),
            jax.ShapeDtypeStruct((_T, _K), jnp.float32),
            jax.ShapeDtypeStruct((_NT, 1), jnp.int32),
            jax.ShapeDtypeStruct((_NT, 1), jnp.int32),
            jax.ShapeDtypeStruct((_NT, 1), jnp.int32),
        ),
    )(hs2d, gate_w)


_FT = 2
_F = _DFF // _FT


def _ffn_body(te_ref, act_ref, xi_ref, x_ref, w1_ref, w3_ref, w2_ref, wcol_ref,
              out_ref):
    i = pl.program_id(1)

    @pl.when(act_ref[i] != 0)
    def _():
        x = x_ref[...]
        a = lax.dot_general(x, w1_ref[0], (((1,), (1,)), ((), ())),
                            preferred_element_type=jnp.float32)
        b = lax.dot_general(x, w3_ref[0], (((1,), (1,)), ((), ())),
                            preferred_element_type=jnp.float32)
        h = (a * lax.logistic(a)) * b
        contrib = lax.dot_general(h, w2_ref[0], (((1,), (1,)), ((), ())),
                                  preferred_element_type=jnp.float32)
        out_ref[...] = contrib * wcol_ref[...]


def _grouped_ffn(x_sorted, w1, w3, w2, wcol, te, act, xi):
    def x_map(f, i, te_r, act_r, xi_r):
        return (xi_r[i], 0)

    def w13_map(f, i, te_r, act_r, xi_r):
        return (te_r[i], f, 0)

    def w2_map(f, i, te_r, act_r, xi_r):
        return (te_r[i], 0, f)

    def out_map(f, i, te_r, act_r, xi_r):
        return (f * _NT + i, 0)

    grid_spec = pltpu.PrefetchScalarGridSpec(
        num_scalar_prefetch=3,
        grid=(_FT, _NT),
        in_specs=[
            pl.BlockSpec((_M, _D), x_map),
            pl.BlockSpec((1, _F, _D), w13_map),
            pl.BlockSpec((1, _F, _D), w13_map),
            pl.BlockSpec((1, _D, _F), w2_map),
            pl.BlockSpec((_M, 1), x_map),
        ],
        out_specs=pl.BlockSpec((_M, _D), out_map),
    )
    return pl.pallas_call(
        _ffn_body,
        grid_spec=grid_spec,
        out_shape=jax.ShapeDtypeStruct((_FT * _P, _D), jnp.float32),
    )(te, act, xi, x_sorted, w1, w3, w2, wcol)


_NC = 2
_NS = 16
_NW = _NC * _NS
_DC = 16


def _sc_dispatch(table, p0, p1):
    toks_per_w = _T // _NW
    n_chunks = toks_per_w // _DC
    mesh = plsc.VectorSubcoreMesh(core_axis_name="c", subcore_axis_name="s")

    @functools.partial(
        pl.kernel,
        out_type=jax.ShapeDtypeStruct((_P, _D), jnp.float32),
        mesh=mesh,
        scratch_types=[
            pltpu.VMEM((_DC,), jnp.int32),
            pltpu.VMEM((_DC,), jnp.int32),
            pltpu.VMEM((_DC,), jnp.int32),
            pltpu.VMEM((_DC,), jnp.int32),
            pltpu.VMEM((_DC, _D), jnp.float32),
            pltpu.VMEM((_DC, _D), jnp.float32),
            pltpu.SemaphoreType.DMA,
            pltpu.SemaphoreType.DMA,
        ],
    )
    def k(table_hbm, p0_hbm, p1_hbm, out_hbm, i0a, i0b, i1a, i1b, ra, rb,
          sem_r, sem_s):
        wid = lax.axis_index("s") * _NC + lax.axis_index("c")
        base = wid * toks_per_w
        idx0 = (i0a, i0b)
        idx1 = (i1a, i1b)
        rbuf = (ra, rb)

        def start_load(j):
            off = base + j * _DC
            sl = pl.ds(off, _DC)
            return (
                pltpu.async_copy(p0_hbm.at[sl], idx0[j % 2], sem_r),
                pltpu.async_copy(p1_hbm.at[sl], idx1[j % 2], sem_r),
                pltpu.async_copy(table_hbm.at[sl], rbuf[j % 2], sem_r),
            )

        def start_scatter(j):
            return (
                pltpu.async_copy(rbuf[j % 2], out_hbm.at[idx0[j % 2]], sem_s),
                pltpu.async_copy(rbuf[j % 2], out_hbm.at[idx1[j % 2]], sem_s),
            )

        ld = [None] * n_chunks
        st = [None] * n_chunks
        for j in range(min(2, n_chunks)):
            ld[j] = start_load(j)
        for j in range(n_chunks):
            for c in ld[j]:
                c.wait()
            st[j] = start_scatter(j)
            if j + 2 < n_chunks:
                for c in st[j]:
                    c.wait()
                ld[j + 2] = start_load(j + 2)
        for j in range(max(0, n_chunks - 2), n_chunks):
            for c in st[j]:
                c.wait()

    return k(table, p0, p1)


_CC = 8
_NG = 4


def _sc_combine(ys, idxs):
    toks_per_w = _T // _NW
    n_chunks = toks_per_w // _CC
    mesh = plsc.VectorSubcoreMesh(core_axis_name="c", subcore_axis_name="s")

    @functools.partial(
        pl.kernel,
        out_type=jax.ShapeDtypeStruct((_T, _D), jnp.float32),
        mesh=mesh,
        scratch_types=[
            pltpu.VMEM((toks_per_w,), jnp.int32) for _ in range(_NG)
        ] + [
            pltpu.VMEM((_CC, _D), jnp.float32) for _ in range(2 * _NG)
        ] + [
            pltpu.SemaphoreType.DMA,
            pltpu.SemaphoreType.DMA,
        ],
    )
    def k(ys_hbm, i0, i1, i2, i3, out_hbm, *rest):
        idx_v = rest[:_NG]
        bufs = (rest[_NG:2 * _NG], rest[2 * _NG:3 * _NG])
        sem_g, sem_w = rest[3 * _NG], rest[3 * _NG + 1]
        wid = lax.axis_index("s") * _NC + lax.axis_index("c")
        base = wid * toks_per_w
        for q, ih in enumerate((i0, i1, i2, i3)):
            pltpu.sync_copy(ih.at[pl.ds(base, toks_per_w)], idx_v[q])

        def start_gathers(j):
            sl = pl.ds(j * _CC, _CC)
            return tuple(
                pltpu.async_copy(ys_hbm.at[idx_v[q].at[sl]], bufs[j % 2][q],
                                 sem_g)
                for q in range(_NG))

        g = [None] * n_chunks
        w = [None] * n_chunks
        for j in range(min(2, n_chunks)):
            g[j] = start_gathers(j)
        for j in range(n_chunks):
            for c in g[j]:
                c.wait()
            bb = bufs[j % 2]

            def row(r, c):
                for v in range(_D // 16):
                    sl = pl.ds(v * 16, 16)
                    bb[0][r, sl] = ((bb[0][r, sl] + bb[1][r, sl])
                                    + (bb[2][r, sl] + bb[3][r, sl]))
                return c

            lax.fori_loop(0, _CC, row, 0)
            w[j] = pltpu.async_copy(
                bb[0], out_hbm.at[pl.ds(base + j * _CC, _CC)], sem_w)
            if j + 2 < n_chunks:
                w[j].wait()
                g[j + 2] = start_gathers(j + 2)
        for j in range(max(0, n_chunks - 2), n_chunks):
            w[j].wait()

    return k(ys, *idxs)


def kernel(hidden_states, gate_w, w1, w2, w3):
    batch, seq, d_model = hidden_states.shape
    hs2d = hidden_states.reshape(-1, d_model)

    logits, p0, p1, wts, te, act, xi = _router(hs2d, gate_w)

    posf = jnp.concatenate([p0, p1], axis=1).reshape(-1)
    w_padded = jnp.zeros((_P,), jnp.float32).at[posf].set(wts.reshape(-1))
    wcol = w_padded.reshape(_P, 1)

    x_sorted = _sc_dispatch(hs2d, p0.reshape(-1), p1.reshape(-1))

    ys = _grouped_ffn(x_sorted, w1, w3, w2, wcol,
                      te.reshape(-1), act.reshape(-1), xi.reshape(-1))

    p0f = p0.reshape(-1)
    p1f = p1.reshape(-1)
    final2d = _sc_combine(ys, (p0f, p1f, p0f + _P, p1f + _P))

    return final2d.reshape(batch, seq, d_model), logits


# speedup vs baseline: 1.2405x; 1.2405x over previous
import functools

import jax
import jax.numpy as jnp
from jax import lax
from jax.experimental import pallas as pl
from jax.experimental.pallas import tpu as pltpu
from jax.experimental.pallas import tpu_sc as plsc

_E = 8
_K = 2
_D = 1024
_DFF = 2048
_T = 2048
_A = _T * _K
_M = 128
_NT = _A // _M + _E
_P = _NT * _M
_MF = float(_M)


def _router_body(hs_ref, gw_ref, logits_ref, p0_ref, p1_ref, wts_ref,
                 te_ref, act_ref, xi_ref):
    hs = hs_ref[...]
    gw = gw_ref[...]
    logits = lax.dot_general(hs, gw, (((1,), (1,)), ((), ())),
                             preferred_element_type=jnp.float32)
    logits_ref[...] = logits
    p = jax.nn.softmax(logits, axis=-1)
    iota = lax.broadcasted_iota(jnp.int32, p.shape, 1)
    m1 = jnp.max(p, axis=1, keepdims=True)
    i1 = jnp.min(jnp.where(p == m1, iota, _E), axis=1, keepdims=True)
    p2 = jnp.where(iota == i1, -1.0, p)
    m2 = jnp.max(p2, axis=1, keepdims=True)
    i2 = jnp.min(jnp.where(p2 == m2, iota, _E), axis=1, keepdims=True)
    den = m1 + m2
    wts_ref[...] = jnp.concatenate([m1 / den, m2 / den], axis=1)

    onehot1 = (iota == i1)
    onehot2 = (iota == i2)
    mask = jnp.where(onehot1 | onehot2, 1.0, 0.0)
    csum = mask
    s = 1
    while s < _T:
        shifted = jnp.concatenate(
            [jnp.zeros((s, _E), jnp.float32), csum[:_T - s, :]], axis=0)
        csum = csum + shifted
        s *= 2
    rank = csum - mask
    counts = csum[_T - 1:_T, :]

    tiles = jnp.floor((counts + (_MF - 1.0)) * (1.0 / _MF))
    tend = tiles
    s = 1
    while s < _E:
        shifted = jnp.concatenate(
            [jnp.zeros((1, s), jnp.float32), tend[:, :_E - s]], axis=1)
        tend = tend + shifted
        s *= 2
    tstart = tend - tiles
    padded_off = tstart * _MF

    slot = padded_off + rank
    p0_ref[...] = jnp.sum(jnp.where(onehot1, slot, 0.0), axis=1,
                          keepdims=True).astype(jnp.int32)
    p1_ref[...] = jnp.sum(jnp.where(onehot2, slot, 0.0), axis=1,
                          keepdims=True).astype(jnp.int32)

    total = jnp.sum(jnp.where(iota[0:1, :] == _E - 1, tend, 0.0),
                    axis=1, keepdims=True)
    j_iota = lax.broadcasted_iota(jnp.int32, (_NT, 1), 0).astype(jnp.float32)
    te_raw = jnp.sum(
        jnp.where(tend <= j_iota, 1.0, 0.0), axis=1, keepdims=True)
    act = jnp.where(j_iota < total, 1, 0).astype(jnp.int32)
    last = jnp.maximum(total - 1.0, 0.0)
    te_last = jnp.sum(jnp.where(tend <= last, 1.0, 0.0), axis=1,
                      keepdims=True)
    te = jnp.where(act != 0, jnp.minimum(te_raw, float(_E - 1)), te_last)
    te_ref[...] = te.astype(jnp.int32)
    act_ref[...] = act
    xi_ref[...] = jnp.where(act != 0, j_iota, last).astype(jnp.int32)


def _router(hs2d, gate_w):
    return pl.pallas_call(
        _router_body,
        out_shape=(
            jax.ShapeDtypeStruct((_T, _E), jnp.float32),
            jax.ShapeDtypeStruct((_T, 1), jnp.int32),
            jax.ShapeDtypeStruct((_T, 1), jnp.int32),
            jax.ShapeDtypeStruct((_T, _K), jnp.float32),
            jax.ShapeDtypeStruct((_NT, 1), jnp.int32),
            jax.ShapeDtypeStruct((_NT, 1), jnp.int32),
            jax.ShapeDtypeStruct((_NT, 1), jnp.int32),
        ),
    )(hs2d, gate_w)


def _ffn_body(te_ref, act_ref, xi_ref, x_ref, w1_ref, w3_ref, w2_ref, wcol_ref,
              out_ref):
    i = pl.program_id(0)

    @pl.when(act_ref[i] != 0)
    def _():
        x = x_ref[...]
        a = lax.dot_general(x, w1_ref[0], (((1,), (1,)), ((), ())),
                            preferred_element_type=jnp.float32)
        b = lax.dot_general(x, w3_ref[0], (((1,), (1,)), ((), ())),
                            preferred_element_type=jnp.float32)
        h = (a * lax.logistic(a)) * b
        contrib = lax.dot_general(h, w2_ref[0], (((1,), (1,)), ((), ())),
                                  preferred_element_type=jnp.float32)
        out_ref[...] = contrib * wcol_ref[...]


def _grouped_ffn(x_sorted, w1, w3, w2, wcol, te, act, xi):
    def x_map(i, te_r, act_r, xi_r):
        return (xi_r[i], 0)

    def w_map(i, te_r, act_r, xi_r):
        return (te_r[i], 0, 0)

    def out_map(i, te_r, act_r, xi_r):
        return (i, 0)

    grid_spec = pltpu.PrefetchScalarGridSpec(
        num_scalar_prefetch=3,
        grid=(_NT,),
        in_specs=[
            pl.BlockSpec((_M, _D), x_map),
            pl.BlockSpec((1, _DFF, _D), w_map),
            pl.BlockSpec((1, _DFF, _D), w_map),
            pl.BlockSpec((1, _D, _DFF), w_map),
            pl.BlockSpec((_M, 1), x_map),
        ],
        out_specs=pl.BlockSpec((_M, _D), out_map),
    )
    return pl.pallas_call(
        _ffn_body,
        grid_spec=grid_spec,
        out_shape=jax.ShapeDtypeStruct((_P, _D), jnp.float32),
    )(te, act, xi, x_sorted, w1, w3, w2, wcol)


_NC = 2
_NS = 16
_NW = _NC * _NS
_DC = 16


def _sc_dispatch(table, p0, p1):
    toks_per_w = _T // _NW
    n_chunks = toks_per_w // _DC
    mesh = plsc.VectorSubcoreMesh(core_axis_name="c", subcore_axis_name="s")

    @functools.partial(
        pl.kernel,
        out_type=jax.ShapeDtypeStruct((_P, _D), jnp.float32),
        mesh=mesh,
        scratch_types=[
            pltpu.VMEM((_DC,), jnp.int32),
            pltpu.VMEM((_DC,), jnp.int32),
            pltpu.VMEM((_DC,), jnp.int32),
            pltpu.VMEM((_DC,), jnp.int32),
            pltpu.VMEM((_DC, _D), jnp.float32),
            pltpu.VMEM((_DC, _D), jnp.float32),
            pltpu.SemaphoreType.DMA,
            pltpu.SemaphoreType.DMA,
        ],
    )
    def k(table_hbm, p0_hbm, p1_hbm, out_hbm, i0a, i0b, i1a, i1b, ra, rb,
          sem_r, sem_s):
        wid = lax.axis_index("s") * _NC + lax.axis_index("c")
        base = wid * toks_per_w
        idx0 = (i0a, i0b)
        idx1 = (i1a, i1b)
        rbuf = (ra, rb)

        def start_load(j):
            off = base + j * _DC
            sl = pl.ds(off, _DC)
            return (
                pltpu.async_copy(p0_hbm.at[sl], idx0[j % 2], sem_r),
                pltpu.async_copy(p1_hbm.at[sl], idx1[j % 2], sem_r),
                pltpu.async_copy(table_hbm.at[sl], rbuf[j % 2], sem_r),
            )

        def start_scatter(j):
            return (
                pltpu.async_copy(rbuf[j % 2], out_hbm.at[idx0[j % 2]], sem_s),
                pltpu.async_copy(rbuf[j % 2], out_hbm.at[idx1[j % 2]], sem_s),
            )

        ld = [None] * n_chunks
        st = [None] * n_chunks
        for j in range(min(2, n_chunks)):
            ld[j] = start_load(j)
        for j in range(n_chunks):
            for c in ld[j]:
                c.wait()
            st[j] = start_scatter(j)
            if j + 2 < n_chunks:
                for c in st[j]:
                    c.wait()
                ld[j + 2] = start_load(j + 2)
        for j in range(max(0, n_chunks - 2), n_chunks):
            for c in st[j]:
                c.wait()

    return k(table, p0, p1)


_CC = 16
_NG = 2


def _sc_combine(ys, idxs):
    toks_per_w = _T // _NW
    n_chunks = toks_per_w // _CC
    mesh = plsc.VectorSubcoreMesh(core_axis_name="c", subcore_axis_name="s")

    @functools.partial(
        pl.kernel,
        out_type=jax.ShapeDtypeStruct((_T, _D), jnp.float32),
        mesh=mesh,
        scratch_types=[
            pltpu.VMEM((toks_per_w,), jnp.int32) for _ in range(_NG)
        ] + [
            pltpu.VMEM((_CC, _D), jnp.float32) for _ in range(2 * _NG)
        ] + [
            pltpu.SemaphoreType.DMA,
            pltpu.SemaphoreType.DMA,
        ],
    )
    def k(ys_hbm, i0, i1, out_hbm, *rest):
        idx_v = rest[:_NG]
        bufs = (rest[_NG:2 * _NG], rest[2 * _NG:3 * _NG])
        sem_g, sem_w = rest[3 * _NG], rest[3 * _NG + 1]
        wid = lax.axis_index("s") * _NC + lax.axis_index("c")
        base = wid * toks_per_w
        for q, ih in enumerate((i0, i1)):
            pltpu.sync_copy(ih.at[pl.ds(base, toks_per_w)], idx_v[q])

        def start_gathers(j):
            sl = pl.ds(j * _CC, _CC)
            return tuple(
                pltpu.async_copy(ys_hbm.at[idx_v[q].at[sl]], bufs[j % 2][q],
                                 sem_g)
                for q in range(_NG))

        g = [None] * n_chunks
        w = [None] * n_chunks
        for j in range(min(2, n_chunks)):
            g[j] = start_gathers(j)
        for j in range(n_chunks):
            for c in g[j]:
                c.wait()
            bb = bufs[j % 2]

            def row(r, c):
                for v in range(_D // 16):
                    sl = pl.ds(v * 16, 16)
                    bb[0][r, sl] = bb[0][r, sl] + bb[1][r, sl]
                return c

            lax.fori_loop(0, _CC, row, 0)
            w[j] = pltpu.async_copy(
                bb[0], out_hbm.at[pl.ds(base + j * _CC, _CC)], sem_w)
            if j + 2 < n_chunks:
                w[j].wait()
                g[j + 2] = start_gathers(j + 2)
        for j in range(max(0, n_chunks - 2), n_chunks):
            w[j].wait()

    return k(ys, *idxs)


def kernel(hidden_states, gate_w, w1, w2, w3):
    batch, seq, d_model = hidden_states.shape
    hs2d = hidden_states.reshape(-1, d_model)

    logits, p0, p1, wts, te, act, xi = _router(hs2d, gate_w)

    posf = jnp.concatenate([p0, p1], axis=1).reshape(-1)
    w_padded = jnp.zeros((_P,), jnp.float32).at[posf].set(wts.reshape(-1))
    wcol = w_padded.reshape(_P, 1)

    x_sorted = _sc_dispatch(hs2d, p0.reshape(-1), p1.reshape(-1))

    ys = _grouped_ffn(x_sorted, w1, w3, w2, wcol,
                      te.reshape(-1), act.reshape(-1), xi.reshape(-1))

    final2d = _sc_combine(ys, (p0.reshape(-1), p1.reshape(-1)))

    return final2d.reshape(batch, seq, d_model), logits
